# Initial kernel scaffold; baseline (speedup 1.0000x reference)
import jax
import jax.numpy as jnp
from jax.experimental import pallas as pl


def kernel(x, position_embedding):
    raise NotImplementedError("write your pallas kernel here")


# speedup vs baseline: 3.1724x; 3.1724x over previous
import functools

import jax
import jax.numpy as jnp
from jax import lax
from jax.experimental import pallas as pl
from jax.experimental.pallas import tpu as pltpu
from jax.experimental.pallas import tpu_sc as plsc

_B = 4096 * 200
_D = 64
_NC = 2
_NS = 16
_NW = _NC * _NS
_R = _B // _NW
_C = 128
_STEPS = _R // _C

_mesh = plsc.VectorSubcoreMesh(core_axis_name="c", subcore_axis_name="s")


@functools.partial(
    pl.kernel,
    mesh=_mesh,
    out_type=jax.ShapeDtypeStruct((_B, _D), jnp.float32),
    scratch_types=[
        pltpu.VMEM((1, _C), jnp.int32),
        pltpu.VMEM((_C, _D), jnp.float32),
        pltpu.SemaphoreType.DMA,
    ],
    compiler_params=pltpu.CompilerParams(use_tc_tiling_on_sc=False),
)
def _gather(table_hbm, idx_hbm, out_hbm, idx_v, rows_v, sem):
    wid = lax.axis_index("s") * _NC + lax.axis_index("c")
    base = wid * _R

    def body(g, carry):
        off = base + g * _C
        pltpu.sync_copy(idx_hbm.at[pl.ds(off, _C)], idx_v.at[0])
        pltpu.async_copy(table_hbm.at[idx_v.at[0]], rows_v, sem).wait()
        pltpu.sync_copy(rows_v, out_hbm.at[pl.ds(off, _C)])
        return carry

    lax.fori_loop(0, _STEPS, body, 0)


def kernel(x, position_embedding):
    flat = x.reshape(-1)
    out = _gather(position_embedding, flat)
    return out.reshape(x.shape + (position_embedding.shape[1],))


# speedup vs baseline: 4.2690x; 1.3457x over previous
import functools

import jax
import jax.numpy as jnp
from jax import lax
from jax.experimental import pallas as pl
from jax.experimental.pallas import tpu as pltpu
from jax.experimental.pallas import tpu_sc as plsc

_B = 4096 * 200
_D = 64
_NC = 2
_NS = 16
_NW = _NC * _NS
_R = _B // _NW
_C = 128
_STEPS = _R // _C
_NBUF = 4

_mesh = plsc.VectorSubcoreMesh(core_axis_name="c", subcore_axis_name="s")


@functools.partial(
    pl.kernel,
    mesh=_mesh,
    out_type=jax.ShapeDtypeStruct((_B, _D), jnp.float32),
    scratch_types=[
        pltpu.VMEM((_STEPS, _C), jnp.int32),
        pltpu.VMEM((_NBUF, _C, _D), jnp.float32),
        pltpu.SemaphoreType.DMA((_NBUF,)),
        pltpu.SemaphoreType.DMA((_NBUF,)),
    ],
    compiler_params=pltpu.CompilerParams(use_tc_tiling_on_sc=False),
)
def _gather(table_hbm, idx_hbm, out_hbm, idx_v, rows_v, gsem, ssem):
    wid = lax.axis_index("s") * _NC + lax.axis_index("c")
    cbase = wid * _STEPS

    pltpu.sync_copy(idx_hbm.at[pl.ds(cbase, _STEPS)], idx_v)

    def start_gather(g, b):
        return pltpu.async_copy(table_hbm.at[idx_v.at[g]], rows_v.at[b], gsem.at[b])

    def wait_gather(b):
        pltpu.make_async_copy(table_hbm.at[idx_v.at[0]], rows_v.at[b], gsem.at[b]).wait()

    def start_store(g, b):
        return pltpu.async_copy(rows_v.at[b], out_hbm.at[pl.ds((cbase + g) * _C, _C)], ssem.at[b])

    def wait_store(b):
        pltpu.make_async_copy(rows_v.at[b], out_hbm.at[pl.ds(0, _C)], ssem.at[b]).wait()

    for b in range(_NBUF):
        start_gather(b, b)

    @pl.loop(0, _STEPS - _NBUF, step=_NBUF)
    def _body(gbase):
        for b in range(_NBUF):
            g = gbase + b
            wait_gather(b)
            start_store(g, b)
            wait_store(b)
            start_gather(g + _NBUF, b)

    for b in range(_NBUF):
        wait_gather(b)
        start_store(_STEPS - _NBUF + b, b)
    for b in range(_NBUF):
        wait_store(b)


def kernel(x, position_embedding):
    flat = x.reshape(-1, _C)
    out = _gather(position_embedding, flat)
    return out.reshape(x.shape + (position_embedding.shape[1],))


# speedup vs baseline: 4.2773x; 1.0019x over previous
import functools

import jax
import jax.numpy as jnp
from jax import lax
from jax.experimental import pallas as pl
from jax.experimental.pallas import tpu as pltpu
from jax.experimental.pallas import tpu_sc as plsc

_B = 4096 * 200
_D = 64
_NC = 2
_NS = 16
_NW = _NC * _NS
_R = _B // _NW
_C = 128
_STEPS = _R // _C
_NBUF = 8
_GA = 5

_mesh = plsc.VectorSubcoreMesh(core_axis_name="c", subcore_axis_name="s")


@functools.partial(
    pl.kernel,
    mesh=_mesh,
    out_type=jax.ShapeDtypeStruct((_B, _D), jnp.float32),
    scratch_types=[
        pltpu.VMEM((_STEPS, _C), jnp.int32),
        pltpu.VMEM((_NBUF, _C, _D), jnp.float32),
        pltpu.SemaphoreType.DMA((_NBUF,)),
        pltpu.SemaphoreType.DMA((_NBUF,)),
    ],
    compiler_params=pltpu.CompilerParams(use_tc_tiling_on_sc=False),
)
def _gather(table_hbm, idx_hbm, out_hbm, idx_v, rows_v, gsem, ssem):
    wid = lax.axis_index("s") * _NC + lax.axis_index("c")
    cbase = wid * _STEPS

    pltpu.sync_copy(idx_hbm.at[pl.ds(cbase, _STEPS)], idx_v)

    def start_gather(g, b):
        pltpu.async_copy(table_hbm.at[idx_v.at[g]], rows_v.at[b], gsem.at[b])

    def wait_gather(b):
        pltpu.make_async_copy(table_hbm.at[idx_v.at[0]], rows_v.at[b], gsem.at[b]).wait()

    def start_store(g, b):
        pltpu.async_copy(rows_v.at[b], out_hbm.at[pl.ds((cbase + g) * _C, _C)], ssem.at[b])

    def wait_store(b):
        pltpu.make_async_copy(rows_v.at[b], out_hbm.at[pl.ds(0, _C)], ssem.at[b]).wait()

    def step(p, b, bq, do_wait_store, do_refill):
        wait_gather(b)
        start_store(p, b)
        if do_refill:
            if do_wait_store:
                wait_store(bq)
            start_gather(p + _GA, bq)

    for g in range(_GA):
        start_gather(g, g % _NBUF)

    for p in range(_NBUF):
        step(p, p % _NBUF, (p + _GA) % _NBUF, p + _GA >= _NBUF, True)

    _steady_hi = ((_STEPS - _GA) // _NBUF) * _NBUF

    @pl.loop(_NBUF, _steady_hi, step=_NBUF)
    def _body(pbase):
        for b in range(_NBUF):
            step(pbase + b, b, (b + _GA) % _NBUF, True, True)

    for p in range(_steady_hi, _STEPS):
        step(p, p % _NBUF, (p + _GA) % _NBUF, True, p + _GA < _STEPS)

    for b in range(_NBUF):
        wait_store(b)


def kernel(x, position_embedding):
    flat = x.reshape(-1, _C)
    out = _gather(position_embedding, flat)
    return out.reshape(x.shape + (position_embedding.shape[1],))
